# baseline (speedup 1.0000x reference)
import functools

import jax
import jax.numpy as jnp
from jax import lax
from jax.experimental import pallas as pl
from jax.experimental.pallas import tpu as pltpu
from jax.experimental.pallas import tpu_sc as plsc

_NC = 2
_NS = 16
_NW = _NC * _NS
_K = 64


def _sc_body(rows, seq_len, dim, in_hbm, emb_hbm, out_hbm, idx_v, buf_v, sem):
    wid = lax.axis_index("s") * _NC + lax.axis_index("c")
    rows_per_w = rows // _NW
    base = wid * rows_per_w
    ebase = lax.rem(base, seq_len)
    for c in range(rows_per_w // _K):
        cbase = base + c * _K
        for j in range(_K // 16):
            idx_v[pl.ds(j * 16, 16)] = (cbase + j * 16) + lax.iota(jnp.int32, 16)
        pltpu.sync_copy(emb_hbm.at[pl.ds(ebase + c * _K, _K)], buf_v)
        pltpu.async_copy(in_hbm.at[idx_v], buf_v, sem, add=True).wait()
        pltpu.sync_copy(buf_v, out_hbm.at[pl.ds(cbase, _K)])


def kernel(inputs, embeddings):
    batch, seq_len, dim = inputs.shape
    rows = batch * seq_len
    in_flat = inputs.reshape(rows, dim)
    pos = embeddings[:seq_len]
    mesh = plsc.VectorSubcoreMesh(core_axis_name="c", subcore_axis_name="s")
    k = pl.kernel(
        functools.partial(_sc_body, rows, seq_len, dim),
        out_type=jax.ShapeDtypeStruct((rows, dim), inputs.dtype),
        mesh=mesh,
        scratch_types=[
            pltpu.VMEM((_K,), jnp.int32),
            pltpu.VMEM((_K, dim), jnp.float32),
            pltpu.SemaphoreType.DMA,
        ],
    )
    return k(in_flat, pos).reshape(batch, seq_len, dim)


# speedup vs baseline: 1.0520x; 1.0520x over previous
import functools

import jax
import jax.numpy as jnp
from jax import lax
from jax.experimental import pallas as pl
from jax.experimental.pallas import tpu as pltpu
from jax.experimental.pallas import tpu_sc as plsc

_NC = 2
_NS = 16
_NW = _NC * _NS
_K = 32
_NBUF = 3


def _sc_body(rows, seq_len, dim, in_hbm, emb_hbm, out_hbm,
             idx_v, bufs, asems, wsems):
    wid = lax.axis_index("s") * _NC + lax.axis_index("c")
    rows_per_w = rows // _NW
    base = wid * rows_per_w
    ebase = lax.rem(base, seq_len)
    nchunks = rows_per_w // _K

    adescs = [None] * nchunks
    wdescs = [None] * nchunks
    for c in range(nchunks):
        s = c % _NBUF
        if c >= _NBUF:
            wdescs[c - _NBUF].wait()
        for j in range(_K // 16):
            idx_v[s][pl.ds(j * 16, 16)] = (base + c * _K + j * 16) + lax.iota(
                jnp.int32, 16)
        pltpu.sync_copy(emb_hbm.at[pl.ds(ebase + c * _K, _K)], bufs[s])
        adescs[c] = pltpu.async_copy(in_hbm.at[idx_v[s]], bufs[s], asems[s],
                                     add=True)
        if c >= 1:
            p = c - 1
            adescs[p].wait()
            wdescs[p] = pltpu.async_copy(
                bufs[p % _NBUF], out_hbm.at[pl.ds(base + p * _K, _K)],
                wsems[p % _NBUF])
    last = nchunks - 1
    adescs[last].wait()
    wdescs[last] = pltpu.async_copy(
        bufs[last % _NBUF], out_hbm.at[pl.ds(base + last * _K, _K)],
        wsems[last % _NBUF])
    for c in range(max(0, nchunks - _NBUF), nchunks):
        wdescs[c].wait()


def kernel(inputs, embeddings):
    batch, seq_len, dim = inputs.shape
    rows = batch * seq_len
    in_flat = inputs.reshape(rows, dim)
    pos = embeddings[:seq_len]
    mesh = plsc.VectorSubcoreMesh(core_axis_name="c", subcore_axis_name="s")
    k = pl.kernel(
        functools.partial(_sc_body, rows, seq_len, dim),
        out_type=jax.ShapeDtypeStruct((rows, dim), inputs.dtype),
        mesh=mesh,
        scratch_types=[
            [pltpu.VMEM((_K,), jnp.int32) for _ in range(_NBUF)],
            [pltpu.VMEM((_K, dim), jnp.float32) for _ in range(_NBUF)],
            [pltpu.SemaphoreType.DMA for _ in range(_NBUF)],
            [pltpu.SemaphoreType.DMA for _ in range(_NBUF)],
        ],
    )
    return k(in_flat, pos).reshape(batch, seq_len, dim)
